# baseline (speedup 1.0000x reference)
import functools

import jax
import jax.numpy as jnp
from jax import lax
from jax.experimental import pallas as pl
from jax.experimental.pallas import tpu as pltpu
from jax.experimental.pallas import tpu_sc as plsc

_NC = 2
_NS = 16
_K = 128

_N = 10000
_NROWS = 10008
_PAD_DST = _N
_ZR_HI = 632
_ZR_LO = _NROWS - 15 * _ZR_HI
_OR_HI = 632
_OR_LO = _N - 15 * _OR_HI


_NBUF = 3
_NIDX = 6


@functools.lru_cache(maxsize=None)
def _make_sc_accum(n_chunks, table_rows):
    assert n_chunks % _NIDX == 0
    mesh = plsc.VectorSubcoreMesh(core_axis_name="c", subcore_axis_name="s")

    scratch = (
        [pltpu.VMEM((_K,), jnp.int32) for _ in range(_NIDX)]
        + [pltpu.VMEM((_K,), jnp.int32) for _ in range(_NIDX)]
        + [pltpu.SemaphoreType.DMA for _ in range(_NIDX)]
        + [pltpu.SemaphoreType.DMA for _ in range(_NIDX)]
        + [pltpu.VMEM((_K, 128), jnp.float32) for _ in range(_NBUF)]
        + [pltpu.SemaphoreType.DMA for _ in range(_NBUF)]
        + [pltpu.SemaphoreType.DMA for _ in range(_NBUF)]
        + [pltpu.VMEM_SHARED((_NROWS, 128), jnp.float32)]
    )

    @functools.partial(
        pl.kernel,
        out_type=jax.ShapeDtypeStruct((_NC, _N, 128), jnp.float32),
        mesh=mesh,
        scratch_types=scratch,
    )
    def sc_accum(table, keys, dsts, zeros, out, *refs):
        kv = list(refs[0:_NIDX])
        dv = list(refs[_NIDX:2 * _NIDX])
        isk = list(refs[2 * _NIDX:3 * _NIDX])
        isd = list(refs[3 * _NIDX:4 * _NIDX])
        rest = refs[4 * _NIDX:]
        rv = list(rest[0:_NBUF])
        gsem = list(rest[_NBUF:2 * _NBUF])
        ssem = list(rest[2 * _NBUF:3 * _NBUF])
        acc = rest[3 * _NBUF]
        c = lax.axis_index("c")
        s = lax.axis_index("s")

        @pl.when(s < _NS - 1)
        def _zero_hi():
            pltpu.sync_copy(zeros.at[pl.ds(s * _ZR_HI, _ZR_HI)],
                            acc.at[pl.ds(s * _ZR_HI, _ZR_HI)])

        @pl.when(s == _NS - 1)
        def _zero_lo():
            pltpu.sync_copy(zeros.at[pl.ds(15 * _ZR_HI, _ZR_LO)],
                            acc.at[pl.ds(15 * _ZR_HI, _ZR_LO)])

        plsc.subcore_barrier()

        tile_base = (c * _NS + s) * n_chunks * _K

        for i in range(5):
            off = tile_base + i * _K
            pltpu.async_copy(keys.at[pl.ds(off, _K)], kv[i], isk[i])
            pltpu.async_copy(dsts.at[pl.ds(off, _K)], dv[i], isd[i])
        for b in range(2):
            pltpu.make_async_copy(keys.at[pl.ds(tile_base, _K)],
                                  kv[b], isk[b]).wait()
            pltpu.async_copy(table.at[kv[b]], rv[b], gsem[b])

        @pl.loop(0, n_chunks, step=_NIDX)
        def _blk(j):
            for db in range(_NIDX):
                k = j + db
                b = db % _NBUF
                pltpu.make_async_copy(table.at[kv[db]], rv[b], gsem[b]).wait()
                pltpu.make_async_copy(dsts.at[pl.ds(tile_base, _K)],
                                      dv[db], isd[db]).wait()
                pltpu.async_copy(rv[b], acc.at[dv[db]], ssem[b], add=True)

                nxt = k + 2
                bb = (db + 2) % _NBUF
                n6 = (db + 2) % _NIDX

                @pl.when(nxt < n_chunks)
                def _launch():
                    @pl.when(nxt >= _NBUF)
                    def _free():
                        pltpu.make_async_copy(rv[bb], acc.at[dv[n6]],
                                              ssem[bb]).wait()

                    pltpu.make_async_copy(keys.at[pl.ds(tile_base, _K)],
                                          kv[n6], isk[n6]).wait()
                    pltpu.async_copy(table.at[kv[n6]], rv[bb], gsem[bb])

                nd = k + 5
                s6 = (db + 5) % _NIDX

                @pl.when(nd < n_chunks)
                def _issue_idx():
                    off = tile_base + nd * _K
                    pltpu.async_copy(keys.at[pl.ds(off, _K)], kv[s6], isk[s6])
                    pltpu.async_copy(dsts.at[pl.ds(off, _K)], dv[s6], isd[s6])

        for b in range(_NBUF):
            pltpu.make_async_copy(rv[b], acc.at[dv[b]], ssem[b]).wait()

        plsc.subcore_barrier()

        @pl.when(s < _NS - 1)
        def _copy_hi():
            pltpu.sync_copy(acc.at[pl.ds(s * _OR_HI, _OR_HI)],
                            out.at[c, pl.ds(s * _OR_HI, _OR_HI)])

        @pl.when(s == _NS - 1)
        def _copy_lo():
            pltpu.sync_copy(acc.at[pl.ds(15 * _OR_HI, _OR_LO)],
                            out.at[c, pl.ds(15 * _OR_HI, _OR_LO)])

    return sc_accum


def _mm_body(x_ref, w_ref, o_ref):
    o_ref[0] = jnp.dot(x_ref[...], w_ref[0], preferred_element_type=jnp.float32)


def _stacked_mm(x, w):
    g, din, dout = w.shape
    n = x.shape[0]
    bm = 1000
    return pl.pallas_call(
        _mm_body,
        grid=(g, n // bm),
        in_specs=[
            pl.BlockSpec((bm, din), lambda gi, m: (m, 0)),
            pl.BlockSpec((1, din, dout), lambda gi, m: (gi, 0, 0)),
        ],
        out_specs=pl.BlockSpec((1, bm, dout), lambda gi, m: (gi, m, 0)),
        out_shape=jax.ShapeDtypeStruct((g, n, dout), jnp.float32),
    )(x, w)


def _relu_mm_body(a_ref, b_ref, w_ref, o_ref, h1_ref):
    @pl.when(pl.program_id(1) == 0)
    def _build():
        bv = b_ref[...]
        h1_ref[:, 0:128] = jnp.maximum(a_ref[0] + bv[:, 0:128], 0.0)
        h1_ref[:, 128:256] = jnp.maximum(a_ref[1] + bv[:, 128:256], 0.0)

    o_ref[0] = jnp.dot(h1_ref[...], w_ref[0], preferred_element_type=jnp.float32)


def _relu_stacked_mm(agg, b, w):
    g, din, dout = w.shape
    bm = 1000
    return pl.pallas_call(
        _relu_mm_body,
        grid=(_N // bm, g),
        in_specs=[
            pl.BlockSpec((2, bm, 128), lambda m, gi: (0, m, 0)),
            pl.BlockSpec((1, 256), lambda m, gi: (0, 0)),
            pl.BlockSpec((1, din, dout), lambda m, gi: (gi, 0, 0)),
        ],
        out_specs=pl.BlockSpec((1, bm, dout), lambda m, gi: (gi, m, 0)),
        out_shape=jax.ShapeDtypeStruct((g, _N, dout), jnp.float32),
        scratch_shapes=[pltpu.VMEM((bm, 256), jnp.float32)],
    )(agg, b, w)


def _final_body(a_ref, b_ref, wc_ref, bc_ref, o_ref):
    h2 = jnp.maximum(a_ref[0] + a_ref[1] + b_ref[...], 0.0)
    m = jnp.sum(h2, axis=0, keepdims=True) * (1.0 / _N)
    logits = jnp.dot(m, wc_ref[...], preferred_element_type=jnp.float32) + bc_ref[...]
    z = logits - jnp.max(logits, axis=1, keepdims=True)
    e = jnp.exp(z)
    o_ref[...] = e / jnp.sum(e, axis=1, keepdims=True)


def _final(agg, b2, wc, bc):
    c = wc.shape[1]
    return pl.pallas_call(
        _final_body,
        out_shape=jax.ShapeDtypeStruct((1, c), jnp.float32),
    )(agg, b2, wc, bc)


def _round_up(x, m):
    return -(-x // m) * m


def _pad_reshape(arr, per_core, n_chunks, pad_value):
    target = _NS * n_chunks * _K
    arr = jnp.pad(arr, ((0, 0), (0, target - per_core)), constant_values=pad_value)
    return arr.reshape(-1)


def kernel(h, edge_index, rel_types, W1, W1_self, b1, W2, W2_self, b2, Wc, bc):
    h = h.astype(jnp.float32)
    n = h.shape[0]
    nr = W1.shape[0]

    w1_all = jnp.concatenate([W1, W1_self[None]], axis=0)
    w2_all = jnp.concatenate([W2, W2_self[None]], axis=0)

    src = edge_index[0]
    dst = edge_index[1]
    nid = jnp.arange(n, dtype=jnp.int32)
    keys_all = jnp.concatenate([rel_types * n + src, nr * n + nid])
    dst_all = jnp.concatenate([dst, nid])
    e_tot = keys_all.shape[0]

    nch1 = _round_up(-(-e_tot // (_NS * _K)), _NIDX)
    k2 = keys_all * 2
    keys1 = _pad_reshape(jnp.stack([k2, k2 + 1]), e_tot, nch1, 0)
    dst1 = _pad_reshape(jnp.stack([dst_all, dst_all]), e_tot, nch1, _PAD_DST)

    eh = -(-e_tot // 2)
    nch2 = _round_up(-(-eh // (_NS * _K)), _NIDX)
    pad_tail = 2 * eh - e_tot
    keys_h = jnp.pad(keys_all, (0, pad_tail)).reshape(2, eh)
    dst_h = jnp.pad(dst_all, (0, pad_tail), constant_values=_PAD_DST).reshape(2, eh)
    keys2 = _pad_reshape(keys_h, eh, nch2, 0)
    dst2 = _pad_reshape(dst_h, eh, nch2, _PAD_DST)

    zeros = jnp.zeros((_NROWS, 128), jnp.float32)

    proj1 = _stacked_mm(h, w1_all)
    table1 = proj1.reshape((nr + 1) * n * 2, 128)
    agg1 = _make_sc_accum(nch1, table1.shape[0])(table1, keys1, dst1, zeros)

    proj2 = _relu_stacked_mm(agg1, b1.reshape(1, 256), w2_all)
    table2 = proj2.reshape((nr + 1) * n, 128)
    agg2 = _make_sc_accum(nch2, table2.shape[0])(table2, keys2, dst2, zeros)

    return _final(agg2, b2.reshape(1, 128), Wc, bc.reshape(1, Wc.shape[1]))


# speedup vs baseline: 1.4712x; 1.4712x over previous
import functools

import jax
import jax.numpy as jnp
from jax import lax
from jax.experimental import pallas as pl
from jax.experimental.pallas import tpu as pltpu
from jax.experimental.pallas import tpu_sc as plsc

_NC = 2
_NS = 16
_K = 128

_N = 10000
_NROWS = 10008
_PAD_DST = _N
_ZR_HI = 632
_ZR_LO = _NROWS - 15 * _ZR_HI
_OR_HI = 632
_OR_LO = _N - 15 * _OR_HI


_NBUF = 3
_NIDX = 6


@functools.lru_cache(maxsize=None)
def _make_sc_accum(n_chunks, table_rows):
    assert n_chunks % _NBUF == 0
    mesh = plsc.VectorSubcoreMesh(core_axis_name="c", subcore_axis_name="s")

    scratch = (
        [pltpu.VMEM((_K,), jnp.int32) for _ in range(_NBUF)]
        + [pltpu.VMEM((_K,), jnp.int32) for _ in range(_NBUF)]
        + [pltpu.VMEM((_K, 128), jnp.float32) for _ in range(_NBUF)]
        + [pltpu.SemaphoreType.DMA for _ in range(_NBUF)]
        + [pltpu.SemaphoreType.DMA for _ in range(_NBUF)]
        + [pltpu.VMEM_SHARED((_NROWS, 128), jnp.float32)]
    )

    @functools.partial(
        pl.kernel,
        out_type=jax.ShapeDtypeStruct((_NC, _N, 128), jnp.float32),
        mesh=mesh,
        scratch_types=scratch,
    )
    def sc_accum(table, keys, dsts, zeros, out, *refs):
        kv = list(refs[0:_NBUF])
        dv = list(refs[_NBUF:2 * _NBUF])
        rv = list(refs[2 * _NBUF:3 * _NBUF])
        gsem = list(refs[3 * _NBUF:4 * _NBUF])
        ssem = list(refs[4 * _NBUF:5 * _NBUF])
        acc = refs[5 * _NBUF]
        c = lax.axis_index("c")
        s = lax.axis_index("s")

        @pl.when(s < _NS - 1)
        def _zero_hi():
            pltpu.sync_copy(zeros.at[pl.ds(s * _ZR_HI, _ZR_HI)],
                            acc.at[pl.ds(s * _ZR_HI, _ZR_HI)])

        @pl.when(s == _NS - 1)
        def _zero_lo():
            pltpu.sync_copy(zeros.at[pl.ds(15 * _ZR_HI, _ZR_LO)],
                            acc.at[pl.ds(15 * _ZR_HI, _ZR_LO)])

        plsc.subcore_barrier()

        tile_base = (c * _NS + s) * n_chunks * _K

        for b in range(2):
            off = tile_base + b * _K
            pltpu.sync_copy(keys.at[pl.ds(off, _K)], kv[b])
            pltpu.sync_copy(dsts.at[pl.ds(off, _K)], dv[b])
            pltpu.async_copy(table.at[kv[b]], rv[b], gsem[b])

        @pl.loop(0, n_chunks, step=_NBUF)
        def _chunk(j):
            for b in range(_NBUF):
                cur = j + b
                pltpu.make_async_copy(table.at[kv[b]], rv[b], gsem[b]).wait()
                pltpu.async_copy(rv[b], acc.at[dv[b]], ssem[b], add=True)
                bb = (b + 2) % _NBUF
                nxt = cur + 2

                @pl.when(nxt < n_chunks)
                def _prefetch():
                    @pl.when(nxt >= _NBUF)
                    def _free():
                        pltpu.make_async_copy(rv[bb], acc.at[dv[bb]],
                                              ssem[bb]).wait()

                    off = tile_base + nxt * _K
                    pltpu.sync_copy(keys.at[pl.ds(off, _K)], kv[bb])
                    pltpu.sync_copy(dsts.at[pl.ds(off, _K)], dv[bb])
                    pltpu.async_copy(table.at[kv[bb]], rv[bb], gsem[bb])

        for b in range(_NBUF):
            pltpu.make_async_copy(rv[b], acc.at[dv[b]], ssem[b]).wait()

        plsc.subcore_barrier()

        @pl.when(s < _NS - 1)
        def _copy_hi():
            pltpu.sync_copy(acc.at[pl.ds(s * _OR_HI, _OR_HI)],
                            out.at[c, pl.ds(s * _OR_HI, _OR_HI)])

        @pl.when(s == _NS - 1)
        def _copy_lo():
            pltpu.sync_copy(acc.at[pl.ds(15 * _OR_HI, _OR_LO)],
                            out.at[c, pl.ds(15 * _OR_HI, _OR_LO)])

    return sc_accum


def _mm_body(x_ref, w_ref, o_ref):
    o_ref[0] = jnp.dot(x_ref[...], w_ref[0], preferred_element_type=jnp.float32)


def _stacked_mm(x, w):
    g, din, dout = w.shape
    n = x.shape[0]
    bm = 1000
    return pl.pallas_call(
        _mm_body,
        grid=(g, n // bm),
        in_specs=[
            pl.BlockSpec((bm, din), lambda gi, m: (m, 0)),
            pl.BlockSpec((1, din, dout), lambda gi, m: (gi, 0, 0)),
        ],
        out_specs=pl.BlockSpec((1, bm, dout), lambda gi, m: (gi, m, 0)),
        out_shape=jax.ShapeDtypeStruct((g, n, dout), jnp.float32),
    )(x, w)


def _relu_mm_body(a_ref, b_ref, w_ref, o_ref, h1_ref):
    @pl.when(pl.program_id(1) == 0)
    def _build():
        bv = b_ref[...]
        h1_ref[:, 0:128] = jnp.maximum(a_ref[0] + bv[:, 0:128], 0.0)
        h1_ref[:, 128:256] = jnp.maximum(a_ref[1] + bv[:, 128:256], 0.0)

    o_ref[0] = jnp.dot(h1_ref[...], w_ref[0], preferred_element_type=jnp.float32)


def _relu_stacked_mm(agg, b, w):
    g, din, dout = w.shape
    bm = 1000
    return pl.pallas_call(
        _relu_mm_body,
        grid=(_N // bm, g),
        in_specs=[
            pl.BlockSpec((2, bm, 128), lambda m, gi: (0, m, 0)),
            pl.BlockSpec((1, 256), lambda m, gi: (0, 0)),
            pl.BlockSpec((1, din, dout), lambda m, gi: (gi, 0, 0)),
        ],
        out_specs=pl.BlockSpec((1, bm, dout), lambda m, gi: (gi, m, 0)),
        out_shape=jax.ShapeDtypeStruct((g, _N, dout), jnp.float32),
        scratch_shapes=[pltpu.VMEM((bm, 256), jnp.float32)],
    )(agg, b, w)


def _final_body(a_ref, b_ref, wc_ref, bc_ref, o_ref):
    h2 = jnp.maximum(a_ref[0] + a_ref[1] + b_ref[...], 0.0)
    m = jnp.sum(h2, axis=0, keepdims=True) * (1.0 / _N)
    logits = jnp.dot(m, wc_ref[...], preferred_element_type=jnp.float32) + bc_ref[...]
    z = logits - jnp.max(logits, axis=1, keepdims=True)
    e = jnp.exp(z)
    o_ref[...] = e / jnp.sum(e, axis=1, keepdims=True)


def _final(agg, b2, wc, bc):
    c = wc.shape[1]
    return pl.pallas_call(
        _final_body,
        out_shape=jax.ShapeDtypeStruct((1, c), jnp.float32),
    )(agg, b2, wc, bc)


def _round_up(x, m):
    return -(-x // m) * m


def _pad_reshape(arr, per_core, n_chunks, pad_value):
    target = _NS * n_chunks * _K
    arr = jnp.pad(arr, ((0, 0), (0, target - per_core)), constant_values=pad_value)
    return arr.reshape(-1)


def kernel(h, edge_index, rel_types, W1, W1_self, b1, W2, W2_self, b2, Wc, bc):
    h = h.astype(jnp.float32)
    n = h.shape[0]
    nr = W1.shape[0]

    w1_all = jnp.concatenate([W1, W1_self[None]], axis=0)
    w2_all = jnp.concatenate([W2, W2_self[None]], axis=0)

    src = edge_index[0]
    dst = edge_index[1]
    nid = jnp.arange(n, dtype=jnp.int32)
    keys_all = jnp.concatenate([rel_types * n + src, nr * n + nid])
    dst_all = jnp.concatenate([dst, nid])
    e_tot = keys_all.shape[0]

    nch1 = _round_up(-(-e_tot // (_NS * _K)), _NBUF)
    k2 = keys_all * 2
    keys1 = _pad_reshape(jnp.stack([k2, k2 + 1]), e_tot, nch1, 0)
    dst1 = _pad_reshape(jnp.stack([dst_all, dst_all]), e_tot, nch1, _PAD_DST)

    eh = -(-e_tot // 2)
    nch2 = _round_up(-(-eh // (_NS * _K)), _NBUF)
    pad_tail = 2 * eh - e_tot
    keys_h = jnp.pad(keys_all, (0, pad_tail)).reshape(2, eh)
    dst_h = jnp.pad(dst_all, (0, pad_tail), constant_values=_PAD_DST).reshape(2, eh)
    keys2 = _pad_reshape(keys_h, eh, nch2, 0)
    dst2 = _pad_reshape(dst_h, eh, nch2, _PAD_DST)

    zeros = jnp.zeros((_NROWS, 128), jnp.float32)

    proj1 = _stacked_mm(h, w1_all)
    table1 = proj1.reshape((nr + 1) * n * 2, 128)
    agg1 = _make_sc_accum(nch1, table1.shape[0])(table1, keys1, dst1, zeros)

    proj2 = _relu_stacked_mm(agg1, b1.reshape(1, 256), w2_all)
    table2 = proj2.reshape((nr + 1) * n, 128)
    agg2 = _make_sc_accum(nch2, table2.shape[0])(table2, keys2, dst2, zeros)

    return _final(agg2, b2.reshape(1, 128), Wc, bc.reshape(1, Wc.shape[1]))


# speedup vs baseline: 1.5122x; 1.0279x over previous
import functools

import jax
import jax.numpy as jnp
from jax import lax
from jax.experimental import pallas as pl
from jax.experimental.pallas import tpu as pltpu
from jax.experimental.pallas import tpu_sc as plsc

_NC = 2
_NS = 16
_K = 128

_N = 10000
_NROWS = 10008
_PAD_DST = _N
_ZR_HI = 632
_ZR_LO = _NROWS - 15 * _ZR_HI
_OR_HI = 632
_OR_LO = _N - 15 * _OR_HI


_NBUF = 3
_NIDX = 6


@functools.lru_cache(maxsize=None)
def _make_sc_accum(n_chunks, table_rows):
    assert n_chunks % _NBUF == 0
    mesh = plsc.VectorSubcoreMesh(core_axis_name="c", subcore_axis_name="s")

    scratch = (
        [pltpu.VMEM((2, _K), jnp.int32) for _ in range(_NBUF)]
        + [pltpu.VMEM((_K, 128), jnp.float32) for _ in range(_NBUF)]
        + [pltpu.SemaphoreType.DMA for _ in range(_NBUF)]
        + [pltpu.SemaphoreType.DMA for _ in range(_NBUF)]
        + [pltpu.VMEM_SHARED((_NROWS, 128), jnp.float32)]
    )

    @functools.partial(
        pl.kernel,
        out_type=jax.ShapeDtypeStruct((_NC, _N, 128), jnp.float32),
        mesh=mesh,
        scratch_types=scratch,
    )
    def sc_accum(table, comb, zeros, out, *refs):
        cv = list(refs[0:_NBUF])
        rv = list(refs[_NBUF:2 * _NBUF])
        gsem = list(refs[2 * _NBUF:3 * _NBUF])
        ssem = list(refs[3 * _NBUF:4 * _NBUF])
        acc = refs[4 * _NBUF]
        c = lax.axis_index("c")
        s = lax.axis_index("s")

        @pl.when(s < _NS - 1)
        def _zero_hi():
            pltpu.sync_copy(zeros.at[pl.ds(s * _ZR_HI, _ZR_HI)],
                            acc.at[pl.ds(s * _ZR_HI, _ZR_HI)])

        @pl.when(s == _NS - 1)
        def _zero_lo():
            pltpu.sync_copy(zeros.at[pl.ds(15 * _ZR_HI, _ZR_LO)],
                            acc.at[pl.ds(15 * _ZR_HI, _ZR_LO)])

        plsc.subcore_barrier()

        tile_base = (c * _NS + s) * n_chunks

        for b in range(2):
            pltpu.sync_copy(comb.at[tile_base + b], cv[b])
            pltpu.async_copy(table.at[cv[b].at[0]], rv[b], gsem[b])

        @pl.loop(0, n_chunks, step=_NBUF)
        def _chunk(j):
            for b in range(_NBUF):
                cur = j + b
                pltpu.make_async_copy(table.at[cv[b].at[0]], rv[b],
                                      gsem[b]).wait()
                pltpu.async_copy(rv[b], acc.at[cv[b].at[1]], ssem[b], add=True)
                bb = (b + 2) % _NBUF
                nxt = cur + 2

                @pl.when(nxt < n_chunks)
                def _prefetch():
                    @pl.when(nxt >= _NBUF)
                    def _free():
                        pltpu.make_async_copy(rv[bb], acc.at[cv[bb].at[1]],
                                              ssem[bb]).wait()

                    pltpu.sync_copy(comb.at[tile_base + nxt], cv[bb])
                    pltpu.async_copy(table.at[cv[bb].at[0]], rv[bb], gsem[bb])

        for b in range(_NBUF):
            pltpu.make_async_copy(rv[b], acc.at[cv[b].at[1]], ssem[b]).wait()

        plsc.subcore_barrier()

        @pl.when(s < _NS - 1)
        def _copy_hi():
            pltpu.sync_copy(acc.at[pl.ds(s * _OR_HI, _OR_HI)],
                            out.at[c, pl.ds(s * _OR_HI, _OR_HI)])

        @pl.when(s == _NS - 1)
        def _copy_lo():
            pltpu.sync_copy(acc.at[pl.ds(15 * _OR_HI, _OR_LO)],
                            out.at[c, pl.ds(15 * _OR_HI, _OR_LO)])

    return sc_accum


def _mm_body(x_ref, w_ref, o_ref):
    o_ref[0] = jnp.dot(x_ref[...], w_ref[0], preferred_element_type=jnp.float32)


def _stacked_mm(x, w):
    g, din, dout = w.shape
    n = x.shape[0]
    bm = 1000
    return pl.pallas_call(
        _mm_body,
        grid=(g, n // bm),
        in_specs=[
            pl.BlockSpec((bm, din), lambda gi, m: (m, 0)),
            pl.BlockSpec((1, din, dout), lambda gi, m: (gi, 0, 0)),
        ],
        out_specs=pl.BlockSpec((1, bm, dout), lambda gi, m: (gi, m, 0)),
        out_shape=jax.ShapeDtypeStruct((g, n, dout), jnp.float32),
    )(x, w)


def _relu_mm_body(a_ref, b_ref, w_ref, o_ref, h1_ref):
    @pl.when(pl.program_id(1) == 0)
    def _build():
        bv = b_ref[...]
        h1_ref[:, 0:128] = jnp.maximum(a_ref[0] + bv[:, 0:128], 0.0)
        h1_ref[:, 128:256] = jnp.maximum(a_ref[1] + bv[:, 128:256], 0.0)

    o_ref[0] = jnp.dot(h1_ref[...], w_ref[0], preferred_element_type=jnp.float32)


def _relu_stacked_mm(agg, b, w):
    g, din, dout = w.shape
    bm = 1000
    return pl.pallas_call(
        _relu_mm_body,
        grid=(_N // bm, g),
        in_specs=[
            pl.BlockSpec((2, bm, 128), lambda m, gi: (0, m, 0)),
            pl.BlockSpec((1, 256), lambda m, gi: (0, 0)),
            pl.BlockSpec((1, din, dout), lambda m, gi: (gi, 0, 0)),
        ],
        out_specs=pl.BlockSpec((1, bm, dout), lambda m, gi: (gi, m, 0)),
        out_shape=jax.ShapeDtypeStruct((g, _N, dout), jnp.float32),
        scratch_shapes=[pltpu.VMEM((bm, 256), jnp.float32)],
    )(agg, b, w)


def _final_body(a_ref, b_ref, wc_ref, bc_ref, o_ref):
    h2 = jnp.maximum(a_ref[0] + a_ref[1] + b_ref[...], 0.0)
    m = jnp.sum(h2, axis=0, keepdims=True) * (1.0 / _N)
    logits = jnp.dot(m, wc_ref[...], preferred_element_type=jnp.float32) + bc_ref[...]
    z = logits - jnp.max(logits, axis=1, keepdims=True)
    e = jnp.exp(z)
    o_ref[...] = e / jnp.sum(e, axis=1, keepdims=True)


def _final(agg, b2, wc, bc):
    c = wc.shape[1]
    return pl.pallas_call(
        _final_body,
        out_shape=jax.ShapeDtypeStruct((1, c), jnp.float32),
    )(agg, b2, wc, bc)


def _round_up(x, m):
    return -(-x // m) * m


def _pad_reshape(arr, per_core, n_chunks, pad_value):
    target = _NS * n_chunks * _K
    arr = jnp.pad(arr, ((0, 0), (0, target - per_core)), constant_values=pad_value)
    return arr.reshape(-1, _K)


def _combine_idx(keys_rows, dst_rows):
    return jnp.stack([keys_rows, dst_rows], axis=1)


def kernel(h, edge_index, rel_types, W1, W1_self, b1, W2, W2_self, b2, Wc, bc):
    h = h.astype(jnp.float32)
    n = h.shape[0]
    nr = W1.shape[0]

    w1_all = jnp.concatenate([W1, W1_self[None]], axis=0)
    w2_all = jnp.concatenate([W2, W2_self[None]], axis=0)

    src = edge_index[0]
    dst = edge_index[1]
    nid = jnp.arange(n, dtype=jnp.int32)
    keys_all = jnp.concatenate([rel_types * n + src, nr * n + nid])
    dst_all = jnp.concatenate([dst, nid])
    e_tot = keys_all.shape[0]

    nch1 = _round_up(-(-e_tot // (_NS * _K)), _NBUF)
    k2 = keys_all * 2
    keys1 = _pad_reshape(jnp.stack([k2, k2 + 1]), e_tot, nch1, 0)
    dst1 = _pad_reshape(jnp.stack([dst_all, dst_all]), e_tot, nch1, _PAD_DST)

    eh = -(-e_tot // 2)
    nch2 = _round_up(-(-eh // (_NS * _K)), _NBUF)
    pad_tail = 2 * eh - e_tot
    keys_h = jnp.pad(keys_all, (0, pad_tail)).reshape(2, eh)
    dst_h = jnp.pad(dst_all, (0, pad_tail), constant_values=_PAD_DST).reshape(2, eh)
    keys2 = _pad_reshape(keys_h, eh, nch2, 0)
    dst2 = _pad_reshape(dst_h, eh, nch2, _PAD_DST)

    zeros = jnp.zeros((_NROWS, 128), jnp.float32)

    proj1 = _stacked_mm(h, w1_all)
    table1 = proj1.reshape((nr + 1) * n * 2, 128)
    agg1 = _make_sc_accum(nch1, table1.shape[0])(
        table1, _combine_idx(keys1, dst1), zeros)

    proj2 = _relu_stacked_mm(agg1, b1.reshape(1, 256), w2_all)
    table2 = proj2.reshape((nr + 1) * n, 128)
    agg2 = _make_sc_accum(nch2, table2.shape[0])(
        table2, _combine_idx(keys2, dst2), zeros)

    return _final(agg2, b2.reshape(1, 128), Wc, bc.reshape(1, Wc.shape[1]))


# speedup vs baseline: 1.5143x; 1.0013x over previous
import functools

import jax
import jax.numpy as jnp
from jax import lax
from jax.experimental import pallas as pl
from jax.experimental.pallas import tpu as pltpu
from jax.experimental.pallas import tpu_sc as plsc

_NC = 2
_NS = 16
_K = 128

_N = 10000
_NPAD = 64
_NROWS = _N + _NPAD
_PAD_DST = _N
_ZR_HI = 632
_ZR_LO = _NROWS - 15 * _ZR_HI
_OR_HI = 632
_OR_LO = _N - 15 * _OR_HI


_NBUF = 3
_NIDX = 6


@functools.lru_cache(maxsize=None)
def _make_sc_accum(n_chunks, table_rows):
    assert n_chunks % _NBUF == 0
    mesh = plsc.VectorSubcoreMesh(core_axis_name="c", subcore_axis_name="s")

    scratch = (
        [pltpu.VMEM((2, _K), jnp.int32) for _ in range(_NBUF)]
        + [pltpu.VMEM((_K, 128), jnp.float32) for _ in range(_NBUF)]
        + [pltpu.SemaphoreType.DMA for _ in range(_NBUF)]
        + [pltpu.SemaphoreType.DMA for _ in range(_NBUF)]
        + [pltpu.VMEM_SHARED((_NROWS, 128), jnp.float32)]
    )

    @functools.partial(
        pl.kernel,
        out_type=jax.ShapeDtypeStruct((_NC, _N, 128), jnp.float32),
        mesh=mesh,
        scratch_types=scratch,
    )
    def sc_accum(table, comb, zeros, out, *refs):
        cv = list(refs[0:_NBUF])
        rv = list(refs[_NBUF:2 * _NBUF])
        gsem = list(refs[2 * _NBUF:3 * _NBUF])
        ssem = list(refs[3 * _NBUF:4 * _NBUF])
        acc = refs[4 * _NBUF]
        c = lax.axis_index("c")
        s = lax.axis_index("s")

        @pl.when(s < _NS - 1)
        def _zero_hi():
            pltpu.sync_copy(zeros, acc.at[pl.ds(s * _ZR_HI, _ZR_HI)])

        @pl.when(s == _NS - 1)
        def _zero_lo():
            pltpu.sync_copy(zeros.at[pl.ds(0, _ZR_LO)],
                            acc.at[pl.ds(15 * _ZR_HI, _ZR_LO)])

        plsc.subcore_barrier()

        tile_base = (c * _NS + s) * n_chunks

        for b in range(2):
            pltpu.sync_copy(comb.at[tile_base + b], cv[b])
            pltpu.async_copy(table.at[cv[b].at[0]], rv[b], gsem[b])

        @pl.loop(0, n_chunks, step=_NBUF)
        def _chunk(j):
            for b in range(_NBUF):
                cur = j + b
                pltpu.make_async_copy(table.at[cv[b].at[0]], rv[b],
                                      gsem[b]).wait()
                pltpu.async_copy(rv[b], acc.at[cv[b].at[1]], ssem[b], add=True)
                bb = (b + 2) % _NBUF
                nxt = cur + 2

                @pl.when(nxt < n_chunks)
                def _prefetch():
                    @pl.when(nxt >= _NBUF)
                    def _free():
                        pltpu.make_async_copy(rv[bb], acc.at[cv[bb].at[1]],
                                              ssem[bb]).wait()

                    pltpu.sync_copy(comb.at[tile_base + nxt], cv[bb])
                    pltpu.async_copy(table.at[cv[bb].at[0]], rv[bb], gsem[bb])

        for b in range(_NBUF):
            pltpu.make_async_copy(rv[b], acc.at[cv[b].at[1]], ssem[b]).wait()

        plsc.subcore_barrier()

        @pl.when(s < _NS - 1)
        def _copy_hi():
            pltpu.sync_copy(acc.at[pl.ds(s * _OR_HI, _OR_HI)],
                            out.at[c, pl.ds(s * _OR_HI, _OR_HI)])

        @pl.when(s == _NS - 1)
        def _copy_lo():
            pltpu.sync_copy(acc.at[pl.ds(15 * _OR_HI, _OR_LO)],
                            out.at[c, pl.ds(15 * _OR_HI, _OR_LO)])

    return sc_accum


def _mm_body(x_ref, w_ref, o_ref):
    o_ref[0] = jnp.dot(x_ref[...], w_ref[0], preferred_element_type=jnp.float32)


def _stacked_mm(x, w):
    g, din, dout = w.shape
    n = x.shape[0]
    bm = 1000
    return pl.pallas_call(
        _mm_body,
        grid=(g, n // bm),
        in_specs=[
            pl.BlockSpec((bm, din), lambda gi, m: (m, 0)),
            pl.BlockSpec((1, din, dout), lambda gi, m: (gi, 0, 0)),
        ],
        out_specs=pl.BlockSpec((1, bm, dout), lambda gi, m: (gi, m, 0)),
        out_shape=jax.ShapeDtypeStruct((g, n, dout), jnp.float32),
    )(x, w)


def _relu_mm_body(a_ref, b_ref, w_ref, o_ref, h1_ref):
    @pl.when(pl.program_id(1) == 0)
    def _build():
        bv = b_ref[...]
        h1_ref[:, 0:128] = jnp.maximum(a_ref[0] + bv[:, 0:128], 0.0)
        h1_ref[:, 128:256] = jnp.maximum(a_ref[1] + bv[:, 128:256], 0.0)

    o_ref[0] = jnp.dot(h1_ref[...], w_ref[0], preferred_element_type=jnp.float32)


def _relu_stacked_mm(agg, b, w):
    g, din, dout = w.shape
    bm = 1000
    return pl.pallas_call(
        _relu_mm_body,
        grid=(_N // bm, g),
        in_specs=[
            pl.BlockSpec((2, bm, 128), lambda m, gi: (0, m, 0)),
            pl.BlockSpec((1, 256), lambda m, gi: (0, 0)),
            pl.BlockSpec((1, din, dout), lambda m, gi: (gi, 0, 0)),
        ],
        out_specs=pl.BlockSpec((1, bm, dout), lambda m, gi: (gi, m, 0)),
        out_shape=jax.ShapeDtypeStruct((g, _N, dout), jnp.float32),
        scratch_shapes=[pltpu.VMEM((bm, 256), jnp.float32)],
    )(agg, b, w)


def _final_body(a_ref, b_ref, wc_ref, bc_ref, o_ref):
    h2 = jnp.maximum(a_ref[0] + a_ref[1] + b_ref[...], 0.0)
    m = jnp.sum(h2, axis=0, keepdims=True) * (1.0 / _N)
    logits = jnp.dot(m, wc_ref[...], preferred_element_type=jnp.float32) + bc_ref[...]
    z = logits - jnp.max(logits, axis=1, keepdims=True)
    e = jnp.exp(z)
    o_ref[...] = e / jnp.sum(e, axis=1, keepdims=True)


def _final(agg, b2, wc, bc):
    c = wc.shape[1]
    return pl.pallas_call(
        _final_body,
        out_shape=jax.ShapeDtypeStruct((1, c), jnp.float32),
    )(agg, b2, wc, bc)


def _round_up(x, m):
    return -(-x // m) * m


def _pad_reshape(arr, per_core, n_chunks, pad_dst):
    target = _NS * n_chunks * _K
    padlen = target - per_core
    if pad_dst:
        padvals = _PAD_DST + (jnp.arange(padlen, dtype=jnp.int32) % _NPAD)
    else:
        padvals = jnp.zeros((padlen,), jnp.int32)
    arr = jnp.concatenate(
        [arr, jnp.broadcast_to(padvals, (_NC, padlen))], axis=1)
    return arr.reshape(-1, _K)


def _combine_idx(keys_rows, dst_rows):
    return jnp.stack([keys_rows, dst_rows], axis=1)


def kernel(h, edge_index, rel_types, W1, W1_self, b1, W2, W2_self, b2, Wc, bc):
    h = h.astype(jnp.float32)
    n = h.shape[0]
    nr = W1.shape[0]

    w1_all = jnp.concatenate([W1, W1_self[None]], axis=0)
    w2_all = jnp.concatenate([W2, W2_self[None]], axis=0)

    src = edge_index[0]
    dst = edge_index[1]
    nid = jnp.arange(n, dtype=jnp.int32)
    keys_all = jnp.concatenate([rel_types * n + src, nr * n + nid])
    dst_all = jnp.concatenate([dst, nid])
    e_tot = keys_all.shape[0]

    nch1 = _round_up(-(-e_tot // (_NS * _K)), _NBUF)
    k2 = keys_all * 2
    keys1 = _pad_reshape(jnp.stack([k2, k2 + 1]), e_tot, nch1, False)
    dst1 = _pad_reshape(jnp.stack([dst_all, dst_all]), e_tot, nch1, True)

    eh = -(-e_tot // 2)
    nch2 = _round_up(-(-eh // (_NS * _K)), _NBUF)
    pad_tail = 2 * eh - e_tot
    keys_h = jnp.pad(keys_all, (0, pad_tail)).reshape(2, eh)
    dst_h = jnp.pad(dst_all, (0, pad_tail), constant_values=_PAD_DST).reshape(2, eh)
    keys2 = _pad_reshape(keys_h, eh, nch2, False)
    dst2 = _pad_reshape(dst_h, eh, nch2, True)

    zeros = jnp.zeros((_ZR_HI, 128), jnp.float32)

    proj1 = _stacked_mm(h, w1_all)
    table1 = proj1.reshape((nr + 1) * n * 2, 128)
    agg1 = _make_sc_accum(nch1, table1.shape[0])(
        table1, _combine_idx(keys1, dst1), zeros)

    proj2 = _relu_stacked_mm(agg1, b1.reshape(1, 256), w2_all)
    table2 = proj2.reshape((nr + 1) * n, 128)
    agg2 = _make_sc_accum(nch2, table2.shape[0])(
        table2, _combine_idx(keys2, dst2), zeros)

    return _final(agg2, b2.reshape(1, 128), Wc, bc.reshape(1, Wc.shape[1]))
